# Initial kernel scaffold; baseline (speedup 1.0000x reference)
import jax
import jax.numpy as jnp
from jax.experimental import pallas as pl


def kernel(x):
    raise NotImplementedError("write your pallas kernel here")


# speedup vs baseline: 3.3698x; 3.3698x over previous
import functools

import jax
import jax.numpy as jnp
from jax import lax
from jax.experimental import pallas as pl
from jax.experimental.pallas import tpu as pltpu
from jax.experimental.pallas import tpu_sc as plsc

_N_TARGET = 64.0
_NX = 4096
_LANES = 16
_CHUNKS = _NX // _LANES
_K_BISECT = 24


def _sigmoid16(z):
    e = jnp.exp(-jnp.abs(z))
    return jnp.where(z >= 0.0, 1.0 / (1.0 + e), e / (1.0 + e))


def _butterfly(v, op):
    lanes = lax.iota(jnp.int32, _LANES)
    for s in (8, 4, 2, 1):
        v = op(v, v.at[lanes ^ s].get(mode="promise_in_bounds"))
    return v


def _lml_body(x_hbm, y_hbm, nu_hbm, x_v, y_v, nu_v):
    wid = lax.axis_index("s") * 2 + lax.axis_index("c")
    pltpu.sync_copy(x_hbm.at[wid], x_v)

    def minmax_step(i, carry):
        mn, mx = carry
        v = x_v[pl.ds(i * _LANES, _LANES)]
        return jnp.minimum(mn, v), jnp.maximum(mx, v)

    v0 = x_v[pl.ds(0, _LANES)]
    mn, mx = lax.fori_loop(1, _CHUNKS, minmax_step, (v0, v0))
    lo = -_butterfly(mx, jnp.maximum) - 7.0
    hi = -_butterfly(mn, jnp.minimum) + 7.0

    def bisect_step(_, carry):
        lo, hi = carry
        mid = 0.5 * (lo + hi)

        def acc_step(i, acc):
            v = x_v[pl.ds(i * _LANES, _LANES)]
            return acc + _sigmoid16(v + mid)

        acc = lax.fori_loop(0, _CHUNKS, acc_step, jnp.zeros((_LANES,), jnp.float32))
        below = _butterfly(acc, jnp.add) < _N_TARGET
        lo = jnp.where(below, mid, lo)
        hi = jnp.where(below, hi, mid)
        return lo, hi

    lo, hi = lax.fori_loop(0, _K_BISECT, bisect_step, (lo, hi))
    nu = 0.5 * (lo + hi)

    def y_step(i, _):
        v = x_v[pl.ds(i * _LANES, _LANES)]
        y_v[pl.ds(i * _LANES, _LANES)] = _sigmoid16(v + nu)
        return 0

    lax.fori_loop(0, _CHUNKS, y_step, 0)
    pltpu.sync_copy(y_v, y_hbm.at[wid])
    nu_v[...] = nu
    pltpu.sync_copy(nu_v, nu_hbm.at[wid])


@jax.jit
def _lml_sc(x):
    y, nu_pad = pl.kernel(
        _lml_body,
        out_type=[
            jax.ShapeDtypeStruct((32, _NX), jnp.float32),
            jax.ShapeDtypeStruct((32, _LANES), jnp.float32),
        ],
        mesh=plsc.VectorSubcoreMesh(core_axis_name="c", subcore_axis_name="s"),
        scratch_types=[
            pltpu.VMEM((_NX,), jnp.float32),
            pltpu.VMEM((_NX,), jnp.float32),
            pltpu.VMEM((_LANES,), jnp.float32),
        ],
    )(x)
    return y, nu_pad[:, 0]


def kernel(x):
    return _lml_sc(x)


# speedup vs baseline: 4.0858x; 1.2125x over previous
import functools

import jax
import jax.numpy as jnp
from jax import lax
from jax.experimental import pallas as pl
from jax.experimental.pallas import tpu as pltpu
from jax.experimental.pallas import tpu_sc as plsc

_N_TARGET = 64.0
_NX = 4096
_LANES = 16
_CHUNKS = _NX // _LANES
_K_BISECT = 24


def _sigmoid16(z):
    return 1.0 / (1.0 + jnp.exp(-z))


def _butterfly(v, op):
    lanes = lax.iota(jnp.int32, _LANES)
    for s in (8, 4, 2, 1):
        v = op(v, v.at[lanes ^ s].get(mode="promise_in_bounds"))
    return v


def _lml_body(x_hbm, y_hbm, nu_hbm, x_v, y_v, nu_v):
    wid = lax.axis_index("s") * 2 + lax.axis_index("c")
    pltpu.sync_copy(x_hbm.at[wid], x_v)

    def minmax_step(i, carry):
        mn, mx = carry
        v = x_v[pl.ds(i * _LANES, _LANES)]
        return jnp.minimum(mn, v), jnp.maximum(mx, v)

    v0 = x_v[pl.ds(0, _LANES)]
    mn, mx = lax.fori_loop(1, _CHUNKS, minmax_step, (v0, v0), unroll=8)
    lo = -_butterfly(mx, jnp.maximum) - 7.0
    hi = -_butterfly(mn, jnp.minimum) + 7.0

    def bisect_step(_, carry):
        lo, hi = carry
        mid = 0.5 * (lo + hi)

        def acc_step(i, acc):
            v = x_v[pl.ds(i * _LANES, _LANES)]
            return acc + _sigmoid16(v + mid)

        acc = lax.fori_loop(
            0, _CHUNKS, acc_step, jnp.zeros((_LANES,), jnp.float32), unroll=8
        )
        below = _butterfly(acc, jnp.add) < _N_TARGET
        lo = jnp.where(below, mid, lo)
        hi = jnp.where(below, hi, mid)
        return lo, hi

    lo, hi = lax.fori_loop(0, _K_BISECT, bisect_step, (lo, hi))
    nu = 0.5 * (lo + hi)

    def y_step(i, _):
        v = x_v[pl.ds(i * _LANES, _LANES)]
        y_v[pl.ds(i * _LANES, _LANES)] = _sigmoid16(v + nu)
        return 0

    lax.fori_loop(0, _CHUNKS, y_step, 0, unroll=8)
    pltpu.sync_copy(y_v, y_hbm.at[wid])
    nu_v[...] = nu
    pltpu.sync_copy(nu_v, nu_hbm.at[wid])


@jax.jit
def _lml_sc(x):
    y, nu_pad = pl.kernel(
        _lml_body,
        out_type=[
            jax.ShapeDtypeStruct((32, _NX), jnp.float32),
            jax.ShapeDtypeStruct((32, _LANES), jnp.float32),
        ],
        mesh=plsc.VectorSubcoreMesh(core_axis_name="c", subcore_axis_name="s"),
        scratch_types=[
            pltpu.VMEM((_NX,), jnp.float32),
            pltpu.VMEM((_NX,), jnp.float32),
            pltpu.VMEM((_LANES,), jnp.float32),
        ],
    )(x)
    return y, nu_pad[:, 0]


def kernel(x):
    return _lml_sc(x)


# speedup vs baseline: 4.7097x; 1.1527x over previous
import functools

import jax
import jax.numpy as jnp
from jax import lax
from jax.experimental import pallas as pl
from jax.experimental.pallas import tpu as pltpu
from jax.experimental.pallas import tpu_sc as plsc

_N_TARGET = 64.0
_NX = 4096
_LANES = 16
_CHUNKS = _NX // _LANES
_K_RTSAFE = 9


def _sigmoid16(z):
    return 1.0 / (1.0 + jnp.exp(-z))


def _butterfly(v, op):
    lanes = lax.iota(jnp.int32, _LANES)
    for s in (8, 4, 2, 1):
        v = op(v, v.at[lanes ^ s].get(mode="promise_in_bounds"))
    return v


def _lml_body(x_hbm, y_hbm, nu_hbm, x_v, y_v, nu_v):
    wid = lax.axis_index("s") * 2 + lax.axis_index("c")
    pltpu.sync_copy(x_hbm.at[wid], x_v)

    def minmax_step(i, carry):
        mn, mx = carry
        v = x_v[pl.ds(i * _LANES, _LANES)]
        return jnp.minimum(mn, v), jnp.maximum(mx, v)

    v0 = x_v[pl.ds(0, _LANES)]
    mn, mx = lax.fori_loop(1, _CHUNKS, minmax_step, (v0, v0), unroll=8)
    xl = -_butterfly(mx, jnp.maximum) - 7.0
    xh = -_butterfly(mn, jnp.minimum) + 7.0

    rts = 0.5 * (xl + xh)
    dx = xh - xl
    state0 = (xl, xh, rts, dx, dx, rts, jnp.full((_LANES,), jnp.inf, jnp.float32))

    def rtsafe_step(_, carry):
        xl, xh, rts, dx, dxold, best, fbest = carry

        def acc_step(i, carry):
            acc_f, acc_fp = carry
            s = _sigmoid16(x_v[pl.ds(i * _LANES, _LANES)] + rts)
            return acc_f + s, acc_fp + (s - s * s)

        zero = jnp.zeros((_LANES,), jnp.float32)
        acc_f, acc_fp = lax.fori_loop(0, _CHUNKS, acc_step, (zero, zero), unroll=8)
        f = _butterfly(acc_f, jnp.add) - _N_TARGET
        df = _butterfly(acc_fp, jnp.add) + 1e-30
        absf = jnp.abs(f)
        upd = absf < fbest
        best = jnp.where(upd, rts, best)
        fbest = jnp.where(upd, absf, fbest)
        below = f < 0.0
        xl = jnp.where(below, rts, xl)
        xh = jnp.where(below, xh, rts)
        outside = (((rts - xh) * df - f) * ((rts - xl) * df - f)) > 0.0
        slow = 2.0 * absf > jnp.abs(dxold * df)
        bisect = outside | slow
        step = f / df
        half = 0.5 * (xh - xl)
        dxold = dx
        dx = jnp.where(bisect, half, step)
        rts = jnp.where(bisect, xl + half, rts - step)
        return xl, xh, rts, dx, dxold, best, fbest

    nu = lax.fori_loop(0, _K_RTSAFE, rtsafe_step, state0)[5]

    def y_step(i, _):
        v = x_v[pl.ds(i * _LANES, _LANES)]
        y_v[pl.ds(i * _LANES, _LANES)] = _sigmoid16(v + nu)
        return 0

    lax.fori_loop(0, _CHUNKS, y_step, 0, unroll=8)
    pltpu.sync_copy(y_v, y_hbm.at[wid])
    nu_v[...] = nu
    pltpu.sync_copy(nu_v, nu_hbm.at[wid])


@jax.jit
def _lml_sc(x):
    y, nu_pad = pl.kernel(
        _lml_body,
        out_type=[
            jax.ShapeDtypeStruct((32, _NX), jnp.float32),
            jax.ShapeDtypeStruct((32, _LANES), jnp.float32),
        ],
        mesh=plsc.VectorSubcoreMesh(core_axis_name="c", subcore_axis_name="s"),
        scratch_types=[
            pltpu.VMEM((_NX,), jnp.float32),
            pltpu.VMEM((_NX,), jnp.float32),
            pltpu.VMEM((_LANES,), jnp.float32),
        ],
    )(x)
    return y, nu_pad[:, 0]


def kernel(x):
    return _lml_sc(x)
